# baseline (speedup 1.0000x reference)
import jax
import jax.numpy as jnp
from jax import lax
from jax.experimental import pallas as pl
from jax.experimental.pallas import tpu as pltpu
from jax.experimental.pallas import tpu_sc as plsc

B = 16384
V = 1000
D = 1000
DP = 1024
Q = 4
RW = DP // Q
NC = 2
NS = 16
NW = NC * NS
BPW = B // NW
TPW = BPW // 128
NPASS = TPW * Q
SUBROWS = 64
NSUB = 128 // SUBROWS


def _sigmoid_pad_body(x_ref, o_ref):
    o_ref[:, :D] = jax.nn.sigmoid(x_ref[...])
    o_ref[:, D:] = jnp.zeros((V, DP - D), jnp.float32)


def _sigmoid_table(logits):
    return pl.pallas_call(
        _sigmoid_pad_body,
        out_shape=jax.ShapeDtypeStruct((V, DP), jnp.float32),
    )(logits)


def _kc(q):
    return min(RW, D - RW * q)


def _gather_body(s4_hbm, idx_hbm, out_hbm, idx_v, idxq, buf0, buf1,
                 slab0, slab1, si0, si1, so0, so1):
    wid = lax.axis_index("s") * NC + lax.axis_index("c")
    base = wid * BPW
    pltpu.sync_copy(idx_hbm.at[pl.ds(base, BPW)], idx_v)

    riota = lax.iota(jnp.int32, 16)

    for s in range(NPASS * NSUB):
        p, sub = divmod(s, NSUB)
        t, q = divmod(p, Q)
        for g in range(SUBROWS // 16):
            v = idx_v[pl.ds(128 * t + SUBROWS * sub + 16 * g, 16)]
            idxq[pl.ds(SUBROWS * s + 16 * g, 16)] = v * Q + q

    bufs = (buf0, buf1)
    slabs = (slab0, slab1)
    sin = (si0, si1)
    sout = (so0, so1)

    def start_in(s, b):
        return pltpu.async_copy(
            s4_hbm.at[idxq.at[pl.ds(SUBROWS * s, SUBROWS)]], bufs[b], sin[b])

    def start_out(p, sl):
        t, q = divmod(p, Q)
        kc = _kc(q)
        return [pltpu.async_copy(
            slabs[sl].at[pl.ds(0, kc)],
            out_hbm.at[pl.ds(RW * q, kc),
                       pl.ds(128 * (wid * TPW + t), 128)],
            sout[sl])]

    rowvecs = [riota + 16 * g for g in range(SUBROWS // 16)]

    def scatter_sub(s, b):
        p, sub = divmod(s, NSUB)
        q = p % Q
        sl = p % 2
        slab = slabs[sl]
        buf = bufs[b]
        kc = _kc(q)

        def col_body(c, carry):
            col = riota * 0 + c
            for g, rv in enumerate(rowvecs):
                x = plsc.load_gather(buf, [rv, col])
                slab[c, pl.ds(SUBROWS * sub + 16 * g, 16)] = x
            return carry

        lax.fori_loop(0, kc, col_body, 0)

    NTOT = NPASS * NSUB
    h_in = [None, None]
    h_out = [None, None]
    h_in[0] = start_in(0, 0)
    h_in[1] = start_in(1, 1)
    for s in range(NTOT):
        p, sub = divmod(s, NSUB)
        b = s % 2
        if sub == 0 and h_out[p % 2] is not None:
            for h in h_out[p % 2]:
                h.wait()
            h_out[p % 2] = None
        h_in[b].wait()
        scatter_sub(s, b)
        if s + 2 < NTOT:
            h_in[b] = start_in(s + 2, b)
        if sub == NSUB - 1:
            h_out[p % 2] = start_out(p, p % 2)
    for hs in h_out:
        if hs is not None:
            for h in hs:
                h.wait()


_gather = pl.kernel(
    _gather_body,
    out_type=jax.ShapeDtypeStruct((V, B), jnp.float32),
    mesh=plsc.VectorSubcoreMesh(core_axis_name="c", subcore_axis_name="s"),
    scratch_types=(
        [pltpu.VMEM((BPW,), jnp.int32),
         pltpu.VMEM((NPASS * NSUB * SUBROWS,), jnp.int32)]
        + [pltpu.VMEM((SUBROWS, RW), jnp.float32)] * 2
        + [pltpu.VMEM((RW, 128), jnp.float32)] * 2
        + [pltpu.SemaphoreType.DMA] * 4
    ),
    compiler_params=pltpu.CompilerParams(needs_layout_passes=False),
)


@jax.jit
def kernel(indices, logits):
    s = _sigmoid_table(logits)
    s4 = jnp.reshape(s, (Q * V, RW))
    return _gather(s4, indices).T


# speedup vs baseline: 3.3831x; 3.3831x over previous
import jax
import jax.numpy as jnp
from jax import lax
from jax.experimental import pallas as pl
from jax.experimental.pallas import tpu as pltpu
from jax.experimental.pallas import tpu_sc as plsc

B = 16384
V = 1000
D = 1000
DP = 1024
NC = 2
NS = 16
NW = NC * NS
BPW = B // NW
CHUNK = 8
NCH = BPW // CHUNK
LOOKAHEAD = 3


def _sigmoid_pad_body(x_ref, o_ref):
    o_ref[:, :D] = jax.nn.sigmoid(x_ref[...])
    o_ref[:, D:] = jnp.zeros((V, DP - D), jnp.float32)


def _sigmoid_table(logits):
    return pl.pallas_call(
        _sigmoid_pad_body,
        out_shape=jax.ShapeDtypeStruct((V, DP), jnp.float32),
    )(logits)


NBUF = 6


def _gather_body(s_hbm, idx_hbm, out_hbm, idx_v, *rest):
    bufs = rest[:NBUF]
    sin = rest[NBUF:2 * NBUF]
    sout = rest[2 * NBUF:3 * NBUF]
    wid = lax.axis_index("s") * NC + lax.axis_index("c")
    base = wid * BPW
    pltpu.sync_copy(idx_hbm.at[pl.ds(base, BPW)], idx_v)

    def start_in(j, b):
        return pltpu.async_copy(
            s_hbm.at[idx_v.at[pl.ds(j * CHUNK, CHUNK)]], bufs[b], sin[b])

    def start_out(j, b):
        return pltpu.async_copy(
            bufs[b], out_hbm.at[pl.ds(base + j * CHUNK, CHUNK)], sout[b])

    h_in = [None] * NBUF
    h_out = [None] * NBUF
    for j in range(min(LOOKAHEAD, NCH)):
        h_in[j % NBUF] = start_in(j, j % NBUF)
    for j in range(NCH):
        b = j % NBUF
        h_in[b].wait()
        h_out[b] = start_out(j, b)
        nxt = j + LOOKAHEAD
        if nxt < NCH:
            bn = nxt % NBUF
            if h_out[bn] is not None:
                h_out[bn].wait()
                h_out[bn] = None
            h_in[bn] = start_in(nxt, bn)
    for h in h_out:
        if h is not None:
            h.wait()


_gather = pl.kernel(
    _gather_body,
    out_type=jax.ShapeDtypeStruct((B, DP), jnp.float32),
    mesh=plsc.VectorSubcoreMesh(core_axis_name="c", subcore_axis_name="s"),
    scratch_types=(
        [pltpu.VMEM((BPW,), jnp.int32)]
        + [pltpu.VMEM((CHUNK, DP), jnp.float32)] * NBUF
        + [pltpu.SemaphoreType.DMA] * (2 * NBUF)
    ),
)


@jax.jit
def kernel(indices, logits):
    s = _sigmoid_table(logits)
    return _gather(s, indices)[:, :D]


# speedup vs baseline: 3.3930x; 1.0029x over previous
import jax
import jax.numpy as jnp
from jax import lax
from jax.experimental import pallas as pl
from jax.experimental.pallas import tpu as pltpu
from jax.experimental.pallas import tpu_sc as plsc

B = 16384
V = 1000
D = 1000
DP = 1024
NC = 2
NS = 16
NW = NC * NS
BPW = B // NW
CHUNK = 16
NCH = BPW // CHUNK
LOOKAHEAD = 2


def _sigmoid_pad_body(x_ref, o_ref):
    o_ref[:, :D] = jax.nn.sigmoid(x_ref[...])
    o_ref[:, D:] = jnp.zeros((V, DP - D), jnp.float32)


def _sigmoid_table(logits):
    return pl.pallas_call(
        _sigmoid_pad_body,
        out_shape=jax.ShapeDtypeStruct((V, DP), jnp.float32),
    )(logits)


NBUF = 4


def _gather_body(s_hbm, idx_hbm, out_hbm, idx_v, *rest):
    bufs = rest[:NBUF]
    sin = rest[NBUF:2 * NBUF]
    sout = rest[2 * NBUF:3 * NBUF]
    wid = lax.axis_index("s") * NC + lax.axis_index("c")
    base = wid * BPW
    pltpu.sync_copy(idx_hbm.at[pl.ds(base, BPW)], idx_v)

    def start_in(j, b):
        return pltpu.async_copy(
            s_hbm.at[idx_v.at[pl.ds(j * CHUNK, CHUNK)]], bufs[b], sin[b])

    def start_out(j, b):
        return pltpu.async_copy(
            bufs[b], out_hbm.at[pl.ds(base + j * CHUNK, CHUNK)], sout[b])

    h_in = [None] * NBUF
    h_out = [None] * NBUF
    for j in range(min(LOOKAHEAD, NCH)):
        h_in[j % NBUF] = start_in(j, j % NBUF)
    for j in range(NCH):
        b = j % NBUF
        h_in[b].wait()
        h_out[b] = start_out(j, b)
        nxt = j + LOOKAHEAD
        if nxt < NCH:
            bn = nxt % NBUF
            if h_out[bn] is not None:
                h_out[bn].wait()
                h_out[bn] = None
            h_in[bn] = start_in(nxt, bn)
    for h in h_out:
        if h is not None:
            h.wait()


_gather = pl.kernel(
    _gather_body,
    out_type=jax.ShapeDtypeStruct((B, DP), jnp.float32),
    mesh=plsc.VectorSubcoreMesh(core_axis_name="c", subcore_axis_name="s"),
    scratch_types=(
        [pltpu.VMEM((BPW,), jnp.int32)]
        + [pltpu.VMEM((CHUNK, DP), jnp.float32)] * NBUF
        + [pltpu.SemaphoreType.DMA] * (2 * NBUF)
    ),
)


@jax.jit
def kernel(indices, logits):
    s = _sigmoid_table(logits)
    return _gather(s, indices)[:, :D]
